# baseline (speedup 1.0000x reference)
import functools

import jax
import jax.numpy as jnp
from jax import lax
from jax.experimental import pallas as pl
from jax.experimental.pallas import tpu as pltpu
from jax.experimental.pallas import tpu_sc as plsc

B, N1, M1, M2, K = 16, 2048, 1024, 256, 32
_BIG = 1e30
_HI = jax.lax.Precision.DEFAULT


def _dot(a, b):
    return jax.lax.dot_general(a, b, (((1,), (0,)), ((), ())), precision=_HI)


def _dot_t(a, b):
    return jax.lax.dot_general(a, b, (((1,), (1,)), ((), ())), precision=_HI)


def _pairwise_d2(c, x):
    cc = jnp.sum(c * c, axis=-1)
    xx = jnp.sum(x * x, axis=-1)
    return cc[:, None] + xx[None, :] - 2.0 * _dot_t(c, x)


def _topk_mlp_max(d2, g, coff, w2, b2, m, n):
    cols = jax.lax.broadcasted_iota(jnp.int32, (m, n), 1)
    acc0 = jnp.full((m, w2.shape[1]), -_BIG, jnp.float32)

    def body(_, carry):
        d2c, acc = carry
        rmin = jnp.min(d2c, axis=1, keepdims=True)
        cand = jnp.where(d2c == rmin, cols, n)
        amin = jnp.min(cand, axis=1, keepdims=True)
        onehot = (cols == amin)
        d2n = jnp.where(onehot, _BIG, d2c)
        nbg = _dot(onehot.astype(jnp.float32), g)
        h1 = jnp.maximum(nbg - coff, 0.0)
        h2 = jnp.maximum(_dot(h1, w2) + b2, 0.0)
        return d2n, jnp.maximum(acc, h2)

    _, acc = jax.lax.fori_loop(0, K, body, (d2, acc0))
    return acc


def _make_sc_topk_gather(R, N, M, NPTS):
    NC = N // 16
    NG = NC // 16
    NW = 32
    GPW = R // (16 * NW)
    f32, i32 = jnp.float32, jnp.int32
    mesh = plsc.VectorSubcoreMesh(core_axis_name="c", subcore_axis_name="s",
                                  num_cores=2, num_subcores=16)

    @functools.partial(
        pl.kernel,
        out_type=jax.ShapeDtypeStruct((R * K, 32), f32),
        mesh=mesh,
        compiler_params=pltpu.CompilerParams(needs_layout_passes=False,
                                             use_tc_tiling_on_sc=False),
        scratch_types=[
            pltpu.VMEM((16, N), f32),
            pltpu.VMEM((NC * 16,), f32),
            pltpu.VMEM((NG * 16,), f32),
            pltpu.VMEM((16 * K,), i32),
            pltpu.VMEM((16 * K, 32), f32),
            pltpu.VMEM((NPTS, 32), f32),
        ],
    )
    def sc_kernel(d2_hbm, g_hbm, out_hbm, dbuf, chmin, l2, idxf, gbuf, gtab):
        wid = lax.axis_index("s") * 2 + lax.axis_index("c")
        lane = lax.iota(i32, 16)
        zeros = jnp.zeros((16,), i32)
        bat = (wid * GPW * 16) // M
        pltpu.sync_copy(g_hbm.at[pl.ds(bat * NPTS, NPTS)], gtab)

        def group_body(gi, _):
            row0 = (wid * GPW + gi) * 16
            pltpu.sync_copy(d2_hbm.at[pl.ds(row0, 16)], dbuf)

            def chunk_body(c, _):
                col = zeros + c * 16
                mv = plsc.load_gather(dbuf, [lane, col])
                for e in range(1, 16):
                    mv = jnp.minimum(mv, plsc.load_gather(dbuf, [lane, col + e]))
                chmin[pl.ds(c * 16, 16)] = mv
                return 0

            lax.fori_loop(0, NC, chunk_body, 0, unroll=False)

            for q in range(NG):
                mv = chmin[pl.ds(q * 256, 16)]
                for e in range(1, 16):
                    mv = jnp.minimum(mv, chmin[pl.ds(q * 256 + e * 16, 16)])
                l2[pl.ds(q * 16, 16)] = mv

            def ext_body(j, _):
                mv = l2[pl.ds(0, 16)]
                for q in range(1, NG):
                    mv = jnp.minimum(mv, l2[pl.ds(q * 16, 16)])
                g2 = jnp.full((16,), NG, i32)
                for q in range(NG):
                    eq = l2[pl.ds(q * 16, 16)] == mv
                    g2 = jnp.where((g2 == NG) & eq, q, g2)
                c2 = jnp.full((16,), 16, i32)
                cb = g2 * 256 + lane
                for e in range(16):
                    cv = plsc.load_gather(chmin, [cb + e * 16])
                    c2 = jnp.where((c2 == 16) & (cv == mv), e, c2)
                ch = g2 * 16 + c2
                el = jnp.full((16,), 16, i32)
                eb = ch * 16
                for e in range(16):
                    dv = plsc.load_gather(dbuf, [lane, eb + e])
                    el = jnp.where((el == 16) & (dv == mv), e, el)
                n = eb + el
                plsc.store_scatter(idxf, [lane * K + j], n)
                plsc.store_scatter(dbuf, [lane, n],
                                   jnp.full((16,), _BIG, f32))
                mv2 = plsc.load_gather(dbuf, [lane, eb])
                for e in range(1, 16):
                    mv2 = jnp.minimum(mv2, plsc.load_gather(dbuf, [lane, eb + e]))
                plsc.store_scatter(chmin, [ch * 16 + lane], mv2)
                lb = g2 * 256 + lane
                lv = plsc.load_gather(chmin, [lb])
                for e in range(1, 16):
                    lv = jnp.minimum(lv, plsc.load_gather(chmin, [lb + e * 16]))
                plsc.store_scatter(l2, [g2 * 16 + lane], lv)
                return 0

            lax.fori_loop(0, K, ext_body, 0, unroll=False)

            def gat_body(jb, _):
                iv = idxf[pl.ds(jb * 16, 16)]
                rw = jb * 16 + lane
                for c in range(32):
                    cc = zeros + c
                    gv = plsc.load_gather(gtab, [iv, cc])
                    plsc.store_scatter(gbuf, [rw, cc], gv)
                return 0

            lax.fori_loop(0, K, gat_body, 0, unroll=False)
            pltpu.sync_copy(gbuf, out_hbm.at[pl.ds(row0 * K, 16 * K)])
            return 0

        lax.fori_loop(0, GPW, group_body, 0, unroll=False)

    return sc_kernel


def _p1_body(x_ref, c1_ref, cw1_w1_ref, cw1_b1_ref, c1w2_ref, c1b2_ref,
             sa1w1_ref, sa1b1_ref, d2_ref, g_ref, coff_ref):
    x = x_ref[0]
    c = c1_ref[0]
    feat = jnp.maximum(_dot(x, cw1_w1_ref[...]) + cw1_b1_ref[...], 0.0)
    feat = jnp.maximum(_dot(feat, c1w2_ref[...]) + c1b2_ref[...], 0.0)
    w1 = sa1w1_ref[...]
    g_ref[0] = _dot(x, w1[:3]) + _dot(feat, w1[3:])
    coff_ref[0] = _dot(c, w1[:3]) - sa1b1_ref[...]
    d2_ref[0] = _pairwise_d2(c, x)


def _nb_mlp_max(nb, coff, w2, b2, m):
    h1 = jnp.maximum(nb.reshape(m, K, 32) - coff[:, None, :], 0.0)
    h2 = jnp.maximum(_dot(h1.reshape(m * K, 32), w2) + b2, 0.0)
    return jnp.max(h2.reshape(m, K, w2.shape[1]), axis=1)


def _p2_body(nbg_ref, coff_ref, sa1w2_ref, sa1b2_ref,
             c2w1_ref, c2b1_ref, sa2w1_ref, sa2b1_ref, c1_ref, c2_ref,
             d22_ref, g2_ref, coff2_ref):
    feat1 = _nb_mlp_max(nbg_ref[0], coff_ref[0],
                        sa1w2_ref[...], sa1b2_ref[...], M1)
    feat1 = jnp.maximum(_dot(feat1, c2w1_ref[...]) + c2b1_ref[...], 0.0)
    w1 = sa2w1_ref[...]
    c1 = c1_ref[0]
    c2 = c2_ref[0]
    g2_ref[0] = _dot(c1, w1[:3]) + _dot(feat1, w1[3:])
    coff2_ref[0] = _dot(c2, w1[:3]) - sa2b1_ref[...]
    d22_ref[0] = _pairwise_d2(c2, c1)


def _p3_body(nbg2_ref, coff2_ref, sa2w2_ref, sa2b2_ref,
             mlpw_ref, mlpb_ref, out_ref):
    feat2 = _nb_mlp_max(nbg2_ref[0], coff2_ref[0],
                        sa2w2_ref[...], sa2b2_ref[...], M2)
    gmax = jnp.max(feat2, axis=0, keepdims=True)
    out_ref[0] = _dot(gmax, mlpw_ref[...]) + mlpb_ref[...]


def _full(shape):
    nd = len(shape)
    return pl.BlockSpec(shape, lambda b: (0,) * nd)


def _batched(shape):
    nd = len(shape)
    return pl.BlockSpec((1,) + shape, lambda b: (b,) + (0,) * nd)


def kernel(x, c1_w1, c1_b1, c1_w2, c1_b2, sa1_w1, sa1_b1, sa1_w2, sa1_b2,
           c2_w1, c2_b1, sa2_w1, sa2_b1, sa2_w2, sa2_b2, mlp_w, mlp_b):
    f32 = jnp.float32
    centers1 = x[:, ::2][:, :M1]
    stride2 = M1 // M2
    centers2 = centers1[:, ::stride2][:, :M2]

    d2_1, g1, coff1 = pl.pallas_call(
        _p1_body,
        grid=(B,),
        in_specs=[_batched((N1, 3)), _batched((M1, 3)),
                  _full((3, 32)), _full((32,)), _full((32, 32)), _full((32,)),
                  _full((35, 32)), _full((32,))],
        out_specs=[_batched((M1, N1)), _batched((N1, 32)), _batched((M1, 32))],
        out_shape=[jax.ShapeDtypeStruct((B, M1, N1), f32),
                   jax.ShapeDtypeStruct((B, N1, 32), f32),
                   jax.ShapeDtypeStruct((B, M1, 32), f32)],
    )(x, centers1, c1_w1, c1_b1, c1_w2, c1_b2, sa1_w1, sa1_b1)

    nbg1 = _make_sc_topk_gather(B * M1, N1, M1, N1)(
        d2_1.reshape(B * M1, N1), g1.reshape(B * N1, 32))

    d2_2, g2, coff2 = pl.pallas_call(
        _p2_body,
        grid=(B,),
        in_specs=[_batched((M1 * K, 32)), _batched((M1, 32)),
                  _full((32, 32)), _full((32,)),
                  _full((32, 32)), _full((32,)),
                  _full((35, 32)), _full((32,)),
                  _batched((M1, 3)), _batched((M2, 3))],
        out_specs=[_batched((M2, M1)), _batched((M1, 32)), _batched((M2, 32))],
        out_shape=[jax.ShapeDtypeStruct((B, M2, M1), f32),
                   jax.ShapeDtypeStruct((B, M1, 32), f32),
                   jax.ShapeDtypeStruct((B, M2, 32), f32)],
    )(nbg1.reshape(B, M1 * K, 32), coff1, sa1_w2, sa1_b2, c2_w1, c2_b1,
      sa2_w1, sa2_b1, centers1, centers2)

    nbg2 = _make_sc_topk_gather(B * M2, M1, M2, M1)(
        d2_2.reshape(B * M2, M1), g2.reshape(B * M1, 32))

    out = pl.pallas_call(
        _p3_body,
        grid=(B,),
        in_specs=[_batched((M2 * K, 32)), _batched((M2, 32)),
                  _full((32, 128)), _full((128,)),
                  _full((128, 256)), _full((256,))],
        out_specs=_batched((1, 256)),
        out_shape=jax.ShapeDtypeStruct((B, 1, 256), f32),
    )(nbg2.reshape(B, M2 * K, 32), coff2, sa2_w2, sa2_b2, mlp_w, mlp_b)

    return out.reshape(B, 256)


# speedup vs baseline: 1.4567x; 1.4567x over previous
import functools

import jax
import jax.numpy as jnp
from jax import lax
from jax.experimental import pallas as pl
from jax.experimental.pallas import tpu as pltpu
from jax.experimental.pallas import tpu_sc as plsc

B, N1, M1, M2, K = 16, 2048, 1024, 256, 32
_BIG = 1e30
_HI = jax.lax.Precision.DEFAULT


def _dot(a, b):
    return jax.lax.dot_general(a, b, (((1,), (0,)), ((), ())), precision=_HI)


def _dot_t(a, b):
    return jax.lax.dot_general(a, b, (((1,), (1,)), ((), ())), precision=_HI)


def _pairwise_d2(c, x):
    cc = jnp.sum(c * c, axis=-1)
    xx = jnp.sum(x * x, axis=-1)
    return cc[:, None] + xx[None, :] - 2.0 * _dot_t(c, x)


def _topk_mlp_max(d2, g, coff, w2, b2, m, n):
    cols = jax.lax.broadcasted_iota(jnp.int32, (m, n), 1)
    acc0 = jnp.full((m, w2.shape[1]), -_BIG, jnp.float32)

    def body(_, carry):
        d2c, acc = carry
        rmin = jnp.min(d2c, axis=1, keepdims=True)
        cand = jnp.where(d2c == rmin, cols, n)
        amin = jnp.min(cand, axis=1, keepdims=True)
        onehot = (cols == amin)
        d2n = jnp.where(onehot, _BIG, d2c)
        nbg = _dot(onehot.astype(jnp.float32), g)
        h1 = jnp.maximum(nbg - coff, 0.0)
        h2 = jnp.maximum(_dot(h1, w2) + b2, 0.0)
        return d2n, jnp.maximum(acc, h2)

    _, acc = jax.lax.fori_loop(0, K, body, (d2, acc0))
    return acc


def _make_sc_topk_gather(R, N, M, NPTS):
    NC = N // 16
    NG = NC // 16
    NW = 32
    GPW = R // (16 * NW)
    f32, i32 = jnp.float32, jnp.int32
    mesh = plsc.VectorSubcoreMesh(core_axis_name="c", subcore_axis_name="s",
                                  num_cores=2, num_subcores=16)

    @functools.partial(
        pl.kernel,
        out_type=jax.ShapeDtypeStruct((R * K * 32,), f32),
        mesh=mesh,
        compiler_params=pltpu.CompilerParams(needs_layout_passes=False,
                                             use_tc_tiling_on_sc=False),
        scratch_types=[
            pltpu.VMEM((16 * N,), f32),
            pltpu.VMEM((NC * 16,), f32),
            pltpu.VMEM((NG * 16,), f32),
            pltpu.VMEM((16 * K,), i32),
            pltpu.VMEM((16 * K * 32,), f32),
            pltpu.VMEM((NPTS * 32,), f32),
        ],
    )
    def sc_kernel(d2_hbm, g_hbm, out_hbm, dbuf, chmin, l2, idxf, gbuf, gtab):
        wid = lax.axis_index("s") * 2 + lax.axis_index("c")
        lane = lax.iota(i32, 16)
        rowb = lane * N
        bat = (wid * GPW * 16) // M
        pltpu.sync_copy(g_hbm.at[pl.ds(bat * NPTS * 32, NPTS * 32)], gtab)

        def group_body(gi, _):
            row0 = (wid * GPW + gi) * 16
            pltpu.sync_copy(d2_hbm.at[pl.ds(row0 * N, 16 * N)], dbuf)

            def chunk_body(c, _):
                base = rowb + c * 16
                mv = plsc.load_gather(dbuf, [base])
                for e in range(1, 16):
                    mv = jnp.minimum(mv, plsc.load_gather(dbuf, [base + e]))
                chmin[pl.ds(c * 16, 16)] = mv
                return 0

            lax.fori_loop(0, NC, chunk_body, 0, unroll=False)

            for q in range(NG):
                mv = chmin[pl.ds(q * 256, 16)]
                for e in range(1, 16):
                    mv = jnp.minimum(mv, chmin[pl.ds(q * 256 + e * 16, 16)])
                l2[pl.ds(q * 16, 16)] = mv

            def ext_body(j, _):
                mv = l2[pl.ds(0, 16)]
                for q in range(1, NG):
                    mv = jnp.minimum(mv, l2[pl.ds(q * 16, 16)])
                g2 = jnp.full((16,), NG, i32)
                for q in range(NG):
                    eq = l2[pl.ds(q * 16, 16)] == mv
                    g2 = jnp.where((g2 == NG) & eq, q, g2)
                c2 = jnp.full((16,), 16, i32)
                cb = g2 * 256 + lane
                for e in range(16):
                    cv = plsc.load_gather(chmin, [cb + e * 16])
                    c2 = jnp.where((c2 == 16) & (cv == mv), e, c2)
                ch = g2 * 16 + c2
                el = jnp.full((16,), 16, i32)
                eb = rowb + ch * 16
                for e in range(16):
                    dv = plsc.load_gather(dbuf, [eb + e])
                    el = jnp.where((el == 16) & (dv == mv), e, el)
                n = ch * 16 + el
                plsc.store_scatter(idxf, [lane * K + j], n)
                plsc.store_scatter(dbuf, [eb + el],
                                   jnp.full((16,), _BIG, f32))
                mv2 = plsc.load_gather(dbuf, [eb])
                for e in range(1, 16):
                    mv2 = jnp.minimum(mv2, plsc.load_gather(dbuf, [eb + e]))
                plsc.store_scatter(chmin, [ch * 16 + lane], mv2)
                lb = g2 * 256 + lane
                lv = plsc.load_gather(chmin, [lb])
                for e in range(1, 16):
                    lv = jnp.minimum(lv, plsc.load_gather(chmin, [lb + e * 16]))
                plsc.store_scatter(l2, [g2 * 16 + lane], lv)
                return 0

            lax.fori_loop(0, K, ext_body, 0, unroll=False)

            def gat_body(jb, _):
                iv = idxf[pl.ds(jb * 16, 16)]
                for t in range(16):
                    s = iv[t]
                    base = jb * 512 + t * 32
                    gbuf[pl.ds(base, 16)] = gtab[pl.ds(s * 32, 16)]
                    gbuf[pl.ds(base + 16, 16)] = gtab[pl.ds(s * 32 + 16, 16)]
                return 0

            lax.fori_loop(0, K, gat_body, 0, unroll=False)
            pltpu.sync_copy(gbuf, out_hbm.at[pl.ds(row0 * K * 32, 16 * K * 32)])
            return 0

        lax.fori_loop(0, GPW, group_body, 0, unroll=False)

    return sc_kernel


def _p1_body(x_ref, c1_ref, cw1_w1_ref, cw1_b1_ref, c1w2_ref, c1b2_ref,
             sa1w1_ref, sa1b1_ref, d2_ref, g_ref, coff_ref):
    x = x_ref[0]
    c = c1_ref[0]
    feat = jnp.maximum(_dot(x, cw1_w1_ref[...]) + cw1_b1_ref[...], 0.0)
    feat = jnp.maximum(_dot(feat, c1w2_ref[...]) + c1b2_ref[...], 0.0)
    w1 = sa1w1_ref[...]
    g_ref[0] = _dot(x, w1[:3]) + _dot(feat, w1[3:])
    coff_ref[0] = _dot(c, w1[:3]) - sa1b1_ref[...]
    d2_ref[0] = _pairwise_d2(c, x).reshape(M1 * 16, 128)


def _nb_mlp_max(nb, coff, w2, b2, m):
    h1 = jnp.maximum(nb.reshape(m, K, 32) - coff[:, None, :], 0.0)
    h2 = jnp.maximum(_dot(h1.reshape(m * K, 32), w2) + b2, 0.0)
    return jnp.max(h2.reshape(m, K, w2.shape[1]), axis=1)


def _p2_body(nbg_ref, coff_ref, sa1w2_ref, sa1b2_ref,
             c2w1_ref, c2b1_ref, sa2w1_ref, sa2b1_ref, c1_ref, c2_ref,
             d22_ref, g2_ref, coff2_ref):
    feat1 = _nb_mlp_max(nbg_ref[0], coff_ref[0],
                        sa1w2_ref[...], sa1b2_ref[...], M1)
    feat1 = jnp.maximum(_dot(feat1, c2w1_ref[...]) + c2b1_ref[...], 0.0)
    w1 = sa2w1_ref[...]
    c1 = c1_ref[0]
    c2 = c2_ref[0]
    g2_ref[0] = _dot(c1, w1[:3]) + _dot(feat1, w1[3:])
    coff2_ref[0] = _dot(c2, w1[:3]) - sa2b1_ref[...]
    d22_ref[0] = _pairwise_d2(c2, c1).reshape(M2 * 8, 128)


def _p3_body(nbg2_ref, coff2_ref, sa2w2_ref, sa2b2_ref,
             mlpw_ref, mlpb_ref, out_ref):
    feat2 = _nb_mlp_max(nbg2_ref[0], coff2_ref[0],
                        sa2w2_ref[...], sa2b2_ref[...], M2)
    gmax = jnp.max(feat2, axis=0, keepdims=True)
    out_ref[0] = _dot(gmax, mlpw_ref[...]) + mlpb_ref[...]


def _full(shape):
    nd = len(shape)
    return pl.BlockSpec(shape, lambda b: (0,) * nd)


def _batched(shape):
    nd = len(shape)
    return pl.BlockSpec((1,) + shape, lambda b: (b,) + (0,) * nd)


def kernel(x, c1_w1, c1_b1, c1_w2, c1_b2, sa1_w1, sa1_b1, sa1_w2, sa1_b2,
           c2_w1, c2_b1, sa2_w1, sa2_b1, sa2_w2, sa2_b2, mlp_w, mlp_b):
    f32 = jnp.float32
    centers1 = x[:, ::2][:, :M1]
    stride2 = M1 // M2
    centers2 = centers1[:, ::stride2][:, :M2]

    d2_1, g1, coff1 = pl.pallas_call(
        _p1_body,
        grid=(B,),
        in_specs=[_batched((N1, 3)), _batched((M1, 3)),
                  _full((3, 32)), _full((32,)), _full((32, 32)), _full((32,)),
                  _full((35, 32)), _full((32,))],
        out_specs=[_batched((M1 * 16, 128)), _batched((N1, 32)),
                   _batched((M1, 32))],
        out_shape=[jax.ShapeDtypeStruct((B, M1 * 16, 128), f32),
                   jax.ShapeDtypeStruct((B, N1, 32), f32),
                   jax.ShapeDtypeStruct((B, M1, 32), f32)],
    )(x, centers1, c1_w1, c1_b1, c1_w2, c1_b2, sa1_w1, sa1_b1)

    nbg1 = _make_sc_topk_gather(B * M1, N1, M1, N1)(
        d2_1.reshape(B * M1 * N1), g1.reshape(B * N1 * 32))

    d2_2, g2, coff2 = pl.pallas_call(
        _p2_body,
        grid=(B,),
        in_specs=[_batched((M1 * K, 32)), _batched((M1, 32)),
                  _full((32, 32)), _full((32,)),
                  _full((32, 32)), _full((32,)),
                  _full((35, 32)), _full((32,)),
                  _batched((M1, 3)), _batched((M2, 3))],
        out_specs=[_batched((M2 * 8, 128)), _batched((M1, 32)),
                   _batched((M2, 32))],
        out_shape=[jax.ShapeDtypeStruct((B, M2 * 8, 128), f32),
                   jax.ShapeDtypeStruct((B, M1, 32), f32),
                   jax.ShapeDtypeStruct((B, M2, 32), f32)],
    )(nbg1.reshape(B, M1 * K, 32), coff1, sa1_w2, sa1_b2, c2_w1, c2_b1,
      sa2_w1, sa2_b1, centers1, centers2)

    nbg2 = _make_sc_topk_gather(B * M2, M1, M2, M1)(
        d2_2.reshape(B * M2 * M1), g2.reshape(B * M1 * 32))

    out = pl.pallas_call(
        _p3_body,
        grid=(B,),
        in_specs=[_batched((M2 * K, 32)), _batched((M2, 32)),
                  _full((32, 128)), _full((128,)),
                  _full((128, 256)), _full((256,))],
        out_specs=_batched((1, 256)),
        out_shape=jax.ShapeDtypeStruct((B, 1, 256), f32),
    )(nbg2.reshape(B, M2 * K, 32), coff2, sa2_w2, sa2_b2, mlp_w, mlp_b)

    return out.reshape(B, 256)


# speedup vs baseline: 1.5630x; 1.0730x over previous
import functools

import jax
import jax.numpy as jnp
from jax import lax
from jax.experimental import pallas as pl
from jax.experimental.pallas import tpu as pltpu
from jax.experimental.pallas import tpu_sc as plsc

B, N1, M1, M2, K = 16, 2048, 1024, 256, 32
_BIG = 1e30
_HI = jax.lax.Precision.DEFAULT


def _dot(a, b):
    return jax.lax.dot_general(a, b, (((1,), (0,)), ((), ())), precision=_HI)


def _dot_t(a, b):
    return jax.lax.dot_general(a, b, (((1,), (1,)), ((), ())), precision=_HI)


def _pairwise_d2(c, x):
    cc = jnp.sum(c * c, axis=-1)
    xx = jnp.sum(x * x, axis=-1)
    return cc[:, None] + xx[None, :] - 2.0 * _dot_t(c, x)


def _topk_mlp_max(d2, g, coff, w2, b2, m, n):
    cols = jax.lax.broadcasted_iota(jnp.int32, (m, n), 1)
    acc0 = jnp.full((m, w2.shape[1]), -_BIG, jnp.float32)

    def body(_, carry):
        d2c, acc = carry
        rmin = jnp.min(d2c, axis=1, keepdims=True)
        cand = jnp.where(d2c == rmin, cols, n)
        amin = jnp.min(cand, axis=1, keepdims=True)
        onehot = (cols == amin)
        d2n = jnp.where(onehot, _BIG, d2c)
        nbg = _dot(onehot.astype(jnp.float32), g)
        h1 = jnp.maximum(nbg - coff, 0.0)
        h2 = jnp.maximum(_dot(h1, w2) + b2, 0.0)
        return d2n, jnp.maximum(acc, h2)

    _, acc = jax.lax.fori_loop(0, K, body, (d2, acc0))
    return acc


def _make_sc_topk_gather(R, N, M, NPTS):
    NC = N // 16
    NG = NC // 16
    NW = 32
    GPW = R // (16 * NW)
    f32, i32 = jnp.float32, jnp.int32
    mesh = plsc.VectorSubcoreMesh(core_axis_name="c", subcore_axis_name="s",
                                  num_cores=2, num_subcores=16)

    @functools.partial(
        pl.kernel,
        out_type=jax.ShapeDtypeStruct((R * K * 32,), f32),
        mesh=mesh,
        compiler_params=pltpu.CompilerParams(needs_layout_passes=False,
                                             use_tc_tiling_on_sc=False),
        scratch_types=[
            pltpu.VMEM((16 * N,), f32),
            pltpu.VMEM((NC * 16,), f32),
            pltpu.VMEM((NG * 16,), f32),
            pltpu.VMEM((16 * K,), i32),
            pltpu.VMEM((16 * K * 32,), f32),
            pltpu.VMEM((NPTS * 32,), f32),
        ],
    )
    def sc_kernel(d2_hbm, g_hbm, out_hbm, dbuf, chmin, l2, idxf, gbuf, gtab):
        wid = lax.axis_index("s") * 2 + lax.axis_index("c")
        lane = lax.iota(i32, 16)
        rowb = lane * N
        bat = (wid * GPW * 16) // M
        pltpu.sync_copy(g_hbm.at[pl.ds(bat * NPTS * 32, NPTS * 32)], gtab)

        def group_body(gi, _):
            row0 = (wid * GPW + gi) * 16
            pltpu.sync_copy(d2_hbm.at[pl.ds(row0 * N, 16 * N)], dbuf)

            def _tree_min(vals):
                while len(vals) > 1:
                    vals = [jnp.minimum(a, b) for a, b in
                            zip(vals[::2], vals[1::2])]
                return vals[0]

            def chunk_body(c, _):
                base = rowb + c * 16
                chmin[pl.ds(c * 16, 16)] = _tree_min(
                    [plsc.load_gather(dbuf, [base + e]) for e in range(16)])
                return 0

            lax.fori_loop(0, NC, chunk_body, 0, unroll=2)

            for q in range(NG):
                l2[pl.ds(q * 16, 16)] = _tree_min(
                    [chmin[pl.ds(q * 256 + e * 16, 16)] for e in range(16)])

            def ext_body(j, _):
                l2v = [l2[pl.ds(q * 16, 16)] for q in range(NG)]
                mv = _tree_min(l2v)
                g2 = _tree_min([jnp.where(l2v[q] == mv, q, NG)
                                for q in range(NG)])
                cb = g2 * 256 + lane
                cvs = [plsc.load_gather(chmin, [cb + e * 16])
                       for e in range(16)]
                c2 = _tree_min([jnp.where(cvs[e] == mv, e, 16)
                                for e in range(16)])
                ch = g2 * 16 + c2
                eb = rowb + ch * 16
                dvs = [plsc.load_gather(dbuf, [eb + e]) for e in range(16)]
                el = _tree_min([jnp.where(dvs[e] == mv, e, 16)
                                for e in range(16)])
                n = ch * 16 + el
                plsc.store_scatter(idxf, [lane * K + j], n)
                plsc.store_scatter(dbuf, [eb + el],
                                   jnp.full((16,), _BIG, f32))
                mv2 = _tree_min([jnp.where(el == e, _BIG, dvs[e])
                                 for e in range(16)])
                plsc.store_scatter(chmin, [ch * 16 + lane], mv2)
                lv = _tree_min([jnp.where(c2 == e, mv2, cvs[e])
                                for e in range(16)])
                plsc.store_scatter(l2, [g2 * 16 + lane], lv)
                return 0

            lax.fori_loop(0, K, ext_body, 0, unroll=False)

            def gat_body(jb, _):
                iv = idxf[pl.ds(jb * 16, 16)]
                for t in range(16):
                    s = iv[t]
                    base = jb * 512 + t * 32
                    gbuf[pl.ds(base, 16)] = gtab[pl.ds(s * 32, 16)]
                    gbuf[pl.ds(base + 16, 16)] = gtab[pl.ds(s * 32 + 16, 16)]
                return 0

            lax.fori_loop(0, K, gat_body, 0, unroll=False)
            pltpu.sync_copy(gbuf, out_hbm.at[pl.ds(row0 * K * 32, 16 * K * 32)])
            return 0

        lax.fori_loop(0, GPW, group_body, 0, unroll=False)

    return sc_kernel


def _p1_body(x_ref, c1_ref, cw1_w1_ref, cw1_b1_ref, c1w2_ref, c1b2_ref,
             sa1w1_ref, sa1b1_ref, d2_ref, g_ref, coff_ref):
    x = x_ref[0]
    c = c1_ref[0]
    feat = jnp.maximum(_dot(x, cw1_w1_ref[...]) + cw1_b1_ref[...], 0.0)
    feat = jnp.maximum(_dot(feat, c1w2_ref[...]) + c1b2_ref[...], 0.0)
    w1 = sa1w1_ref[...]
    g_ref[0] = _dot(x, w1[:3]) + _dot(feat, w1[3:])
    coff_ref[0] = _dot(c, w1[:3]) - sa1b1_ref[...]
    d2_ref[0] = _pairwise_d2(c, x).reshape(M1 * 16, 128)


def _nb_mlp_max(nb, coff, w2, b2, m):
    h1 = jnp.maximum(nb.reshape(m, K, 32) - coff[:, None, :], 0.0)
    h2 = jnp.maximum(_dot(h1.reshape(m * K, 32), w2) + b2, 0.0)
    return jnp.max(h2.reshape(m, K, w2.shape[1]), axis=1)


def _p2_body(nbg_ref, coff_ref, sa1w2_ref, sa1b2_ref,
             c2w1_ref, c2b1_ref, sa2w1_ref, sa2b1_ref, c1_ref, c2_ref,
             d22_ref, g2_ref, coff2_ref):
    feat1 = _nb_mlp_max(nbg_ref[0], coff_ref[0],
                        sa1w2_ref[...], sa1b2_ref[...], M1)
    feat1 = jnp.maximum(_dot(feat1, c2w1_ref[...]) + c2b1_ref[...], 0.0)
    w1 = sa2w1_ref[...]
    c1 = c1_ref[0]
    c2 = c2_ref[0]
    g2_ref[0] = _dot(c1, w1[:3]) + _dot(feat1, w1[3:])
    coff2_ref[0] = _dot(c2, w1[:3]) - sa2b1_ref[...]
    d22_ref[0] = _pairwise_d2(c2, c1).reshape(M2 * 8, 128)


def _p3_body(nbg2_ref, coff2_ref, sa2w2_ref, sa2b2_ref,
             mlpw_ref, mlpb_ref, out_ref):
    feat2 = _nb_mlp_max(nbg2_ref[0], coff2_ref[0],
                        sa2w2_ref[...], sa2b2_ref[...], M2)
    gmax = jnp.max(feat2, axis=0, keepdims=True)
    out_ref[0] = _dot(gmax, mlpw_ref[...]) + mlpb_ref[...]


def _full(shape):
    nd = len(shape)
    return pl.BlockSpec(shape, lambda b: (0,) * nd)


def _batched(shape):
    nd = len(shape)
    return pl.BlockSpec((1,) + shape, lambda b: (b,) + (0,) * nd)


def kernel(x, c1_w1, c1_b1, c1_w2, c1_b2, sa1_w1, sa1_b1, sa1_w2, sa1_b2,
           c2_w1, c2_b1, sa2_w1, sa2_b1, sa2_w2, sa2_b2, mlp_w, mlp_b):
    f32 = jnp.float32
    centers1 = x[:, ::2][:, :M1]
    stride2 = M1 // M2
    centers2 = centers1[:, ::stride2][:, :M2]

    d2_1, g1, coff1 = pl.pallas_call(
        _p1_body,
        grid=(B,),
        in_specs=[_batched((N1, 3)), _batched((M1, 3)),
                  _full((3, 32)), _full((32,)), _full((32, 32)), _full((32,)),
                  _full((35, 32)), _full((32,))],
        out_specs=[_batched((M1 * 16, 128)), _batched((N1, 32)),
                   _batched((M1, 32))],
        out_shape=[jax.ShapeDtypeStruct((B, M1 * 16, 128), f32),
                   jax.ShapeDtypeStruct((B, N1, 32), f32),
                   jax.ShapeDtypeStruct((B, M1, 32), f32)],
    )(x, centers1, c1_w1, c1_b1, c1_w2, c1_b2, sa1_w1, sa1_b1)

    nbg1 = _make_sc_topk_gather(B * M1, N1, M1, N1)(
        d2_1.reshape(B * M1 * N1), g1.reshape(B * N1 * 32))

    d2_2, g2, coff2 = pl.pallas_call(
        _p2_body,
        grid=(B,),
        in_specs=[_batched((M1 * K, 32)), _batched((M1, 32)),
                  _full((32, 32)), _full((32,)),
                  _full((32, 32)), _full((32,)),
                  _full((35, 32)), _full((32,)),
                  _batched((M1, 3)), _batched((M2, 3))],
        out_specs=[_batched((M2 * 8, 128)), _batched((M1, 32)),
                   _batched((M2, 32))],
        out_shape=[jax.ShapeDtypeStruct((B, M2 * 8, 128), f32),
                   jax.ShapeDtypeStruct((B, M1, 32), f32),
                   jax.ShapeDtypeStruct((B, M2, 32), f32)],
    )(nbg1.reshape(B, M1 * K, 32), coff1, sa1_w2, sa1_b2, c2_w1, c2_b1,
      sa2_w1, sa2_b1, centers1, centers2)

    nbg2 = _make_sc_topk_gather(B * M2, M1, M2, M1)(
        d2_2.reshape(B * M2 * M1), g2.reshape(B * M1 * 32))

    out = pl.pallas_call(
        _p3_body,
        grid=(B,),
        in_specs=[_batched((M2 * K, 32)), _batched((M2, 32)),
                  _full((32, 128)), _full((128,)),
                  _full((128, 256)), _full((256,))],
        out_specs=_batched((1, 256)),
        out_shape=jax.ShapeDtypeStruct((B, 1, 256), f32),
    )(nbg2.reshape(B, M2 * K, 32), coff2, sa2_w2, sa2_b2, mlp_w, mlp_b)

    return out.reshape(B, 256)


# speedup vs baseline: 1.9206x; 1.2288x over previous
import functools

import jax
import jax.numpy as jnp
from jax import lax
from jax.experimental import pallas as pl
from jax.experimental.pallas import tpu as pltpu
from jax.experimental.pallas import tpu_sc as plsc

B, N1, M1, M2, K = 16, 2048, 1024, 256, 32
_BIG = 1e30
_HI = jax.lax.Precision.DEFAULT


def _dot(a, b):
    return jax.lax.dot_general(a, b, (((1,), (0,)), ((), ())), precision=_HI)


def _dot_t(a, b):
    return jax.lax.dot_general(a, b, (((1,), (1,)), ((), ())), precision=_HI)


def _pairwise_d2(c, x):
    cc = jnp.sum(c * c, axis=-1)
    xx = jnp.sum(x * x, axis=-1)
    return cc[:, None] + xx[None, :] - 2.0 * _dot_t(c, x)


def _topk_mlp_max(d2, g, coff, w2, b2, m, n):
    cols = jax.lax.broadcasted_iota(jnp.int32, (m, n), 1)
    acc0 = jnp.full((m, w2.shape[1]), -_BIG, jnp.float32)

    def body(_, carry):
        d2c, acc = carry
        rmin = jnp.min(d2c, axis=1, keepdims=True)
        cand = jnp.where(d2c == rmin, cols, n)
        amin = jnp.min(cand, axis=1, keepdims=True)
        onehot = (cols == amin)
        d2n = jnp.where(onehot, _BIG, d2c)
        nbg = _dot(onehot.astype(jnp.float32), g)
        h1 = jnp.maximum(nbg - coff, 0.0)
        h2 = jnp.maximum(_dot(h1, w2) + b2, 0.0)
        return d2n, jnp.maximum(acc, h2)

    _, acc = jax.lax.fori_loop(0, K, body, (d2, acc0))
    return acc


def _make_sc_topk_gather(R, N, M, NPTS):
    NC = N // 16
    NG = NC // 16
    NW = 32
    GPW = R // (16 * NW)
    f32, i32 = jnp.float32, jnp.int32
    mesh = plsc.VectorSubcoreMesh(core_axis_name="c", subcore_axis_name="s",
                                  num_cores=2, num_subcores=16)

    @functools.partial(
        pl.kernel,
        out_type=jax.ShapeDtypeStruct((R * K * 32,), f32),
        mesh=mesh,
        compiler_params=pltpu.CompilerParams(needs_layout_passes=False,
                                             use_tc_tiling_on_sc=False),
        scratch_types=[
            pltpu.VMEM((16 * N,), f32),
            pltpu.VMEM((NC * 16,), f32),
            pltpu.VMEM((NG * 16,), f32),
            pltpu.VMEM((16 * K,), i32),
            pltpu.VMEM((16 * K * 32,), f32),
            pltpu.VMEM((NPTS * 32,), f32),
        ],
    )
    def sc_kernel(d2_hbm, g_hbm, out_hbm, dbuf, chmin, l2, idxf, gbuf, gtab):
        wid = lax.axis_index("s") * 2 + lax.axis_index("c")
        lane = lax.iota(i32, 16)
        rowb = lane * N
        bat = (wid * GPW * 16) // M
        pltpu.sync_copy(g_hbm.at[pl.ds(bat * NPTS * 32, NPTS * 32)], gtab)

        def group_body(gi, _):
            row0 = (wid * GPW + gi) * 16
            pltpu.sync_copy(d2_hbm.at[pl.ds(row0 * N, 16 * N)], dbuf)

            def _tree_min(vals):
                while len(vals) > 1:
                    vals = [jnp.minimum(a, b) for a, b in
                            zip(vals[::2], vals[1::2])]
                return vals[0]

            def chunk_body(c, _):
                base = rowb + c * 16
                chmin[pl.ds(c * 16, 16)] = _tree_min(
                    [plsc.load_gather(dbuf, [base + e]) for e in range(16)])
                return 0

            lax.fori_loop(0, NC, chunk_body, 0, unroll=2)

            for q in range(NG):
                l2[pl.ds(q * 16, 16)] = _tree_min(
                    [chmin[pl.ds(q * 256 + e * 16, 16)] for e in range(16)])

            def ext_body(j, _):
                l2v = [l2[pl.ds(q * 16, 16)] for q in range(NG)]
                mv = _tree_min(l2v)
                g2 = _tree_min([jnp.where(l2v[q] == mv, q, NG)
                                for q in range(NG)])
                cb = g2 * 256 + lane
                cvs = [plsc.load_gather(chmin, [cb + e * 16])
                       for e in range(16)]
                c2 = _tree_min([jnp.where(cvs[e] == mv, e, 16)
                                for e in range(16)])
                ch = g2 * 16 + c2
                eb = rowb + ch * 16
                dvs = [plsc.load_gather(dbuf, [eb + e]) for e in range(16)]
                el = _tree_min([jnp.where(dvs[e] == mv, e, 16)
                                for e in range(16)])
                n = ch * 16 + el
                plsc.store_scatter(idxf, [lane * K + j], n)
                plsc.store_scatter(dbuf, [eb + el],
                                   jnp.full((16,), _BIG, f32))
                mv2 = _tree_min([jnp.where(el == e, _BIG, dvs[e])
                                 for e in range(16)])
                plsc.store_scatter(chmin, [ch * 16 + lane], mv2)
                lv = _tree_min([jnp.where(c2 == e, mv2, cvs[e])
                                for e in range(16)])
                plsc.store_scatter(l2, [g2 * 16 + lane], lv)
                return 0

            lax.fori_loop(0, K, ext_body, 0, unroll=False)

            def gat_body(jb, _):
                iv = idxf[pl.ds(jb * 16, 16)]
                for t in range(16):
                    s = iv[t]
                    base = jb * 512 + t * 32
                    gbuf[pl.ds(base, 16)] = gtab[pl.ds(s * 32, 16)]
                    gbuf[pl.ds(base + 16, 16)] = gtab[pl.ds(s * 32 + 16, 16)]
                return 0

            lax.fori_loop(0, K, gat_body, 0, unroll=False)
            pltpu.sync_copy(gbuf, out_hbm.at[pl.ds(row0 * K * 32, 16 * K * 32)])
            return 0

        lax.fori_loop(0, GPW, group_body, 0, unroll=False)

    return sc_kernel


def _p1_body(x_ref, c1_ref, cw1_w1_ref, cw1_b1_ref, c1w2_ref, c1b2_ref,
             sa1w1_ref, sa1b1_ref, d2_ref, g_ref, coff_ref):
    x = x_ref[0]
    c = c1_ref[0]
    feat = jnp.maximum(_dot(x, cw1_w1_ref[...]) + cw1_b1_ref[...], 0.0)
    feat = jnp.maximum(_dot(feat, c1w2_ref[...]) + c1b2_ref[...], 0.0)
    w1 = sa1w1_ref[...]
    g_ref[0] = _dot(x, w1[:3]) + _dot(feat, w1[3:])
    coff_ref[0] = _dot(c, w1[:3]) - sa1b1_ref[...]
    d2_ref[0] = _pairwise_d2(c, x).reshape(M1 * 16, 128)


def _nb_mlp_max(nb, coff, w2, b2, m):
    h1 = jnp.maximum(nb.reshape(m, K, 32) - coff[:, None, :], 0.0)
    h2 = jnp.maximum(_dot(h1.reshape(m * K, 32), w2) + b2, 0.0)
    return jnp.max(h2.reshape(m, K, w2.shape[1]), axis=1)


def _p2_body(nbg_ref, coff_ref, sa1w2_ref, sa1b2_ref,
             c2w1_ref, c2b1_ref, sa2w1_ref, sa2b1_ref, c1_ref, c2_ref,
             d22_ref, g2_ref, coff2_ref):
    feat1 = _nb_mlp_max(nbg_ref[0], coff_ref[0],
                        sa1w2_ref[...], sa1b2_ref[...], M1)
    feat1 = jnp.maximum(_dot(feat1, c2w1_ref[...]) + c2b1_ref[...], 0.0)
    w1 = sa2w1_ref[...]
    c1 = c1_ref[0]
    c2 = c2_ref[0]
    g2_ref[0] = _dot(c1, w1[:3]) + _dot(feat1, w1[3:])
    coff2_ref[0] = _dot(c2, w1[:3]) - sa2b1_ref[...]
    d22_ref[0] = _pairwise_d2(c2, c1).reshape(M2 * 8, 128)


def _p3_body(nbg2_ref, coff2_ref, sa2w2_ref, sa2b2_ref,
             mlpw_ref, mlpb_ref, out_ref):
    feat2 = _nb_mlp_max(nbg2_ref[0], coff2_ref[0],
                        sa2w2_ref[...], sa2b2_ref[...], M2)
    gmax = jnp.max(feat2, axis=0, keepdims=True)
    out_ref[0] = _dot(gmax, mlpw_ref[...]) + mlpb_ref[...]


def _full(shape):
    nd = len(shape)
    return pl.BlockSpec(shape, lambda b: (0,) * nd)


def _batched(shape):
    nd = len(shape)
    return pl.BlockSpec((1,) + shape, lambda b: (b,) + (0,) * nd)


def _half_pipeline(x, centers1, centers2, c1_w1, c1_b1, c1_w2, c1_b2,
                   sa1_w1, sa1_b1, sa1_w2, sa1_b2, c2_w1, c2_b1,
                   sa2_w1, sa2_b1, sa2_w2, sa2_b2, mlp_w, mlp_b, nb):
    f32 = jnp.float32

    d2_1, g1, coff1 = pl.pallas_call(
        _p1_body,
        grid=(nb,),
        in_specs=[_batched((N1, 3)), _batched((M1, 3)),
                  _full((3, 32)), _full((32,)), _full((32, 32)), _full((32,)),
                  _full((35, 32)), _full((32,))],
        out_specs=[_batched((M1 * 16, 128)), _batched((N1, 32)),
                   _batched((M1, 32))],
        out_shape=[jax.ShapeDtypeStruct((nb, M1 * 16, 128), f32),
                   jax.ShapeDtypeStruct((nb, N1, 32), f32),
                   jax.ShapeDtypeStruct((nb, M1, 32), f32)],
    )(x, centers1, c1_w1, c1_b1, c1_w2, c1_b2, sa1_w1, sa1_b1)

    nbg1 = _make_sc_topk_gather(nb * M1, N1, M1, N1)(
        d2_1.reshape(nb * M1 * N1), g1.reshape(nb * N1 * 32))

    d2_2, g2, coff2 = pl.pallas_call(
        _p2_body,
        grid=(nb,),
        in_specs=[_batched((M1 * K, 32)), _batched((M1, 32)),
                  _full((32, 32)), _full((32,)),
                  _full((32, 32)), _full((32,)),
                  _full((35, 32)), _full((32,)),
                  _batched((M1, 3)), _batched((M2, 3))],
        out_specs=[_batched((M2 * 8, 128)), _batched((M1, 32)),
                   _batched((M2, 32))],
        out_shape=[jax.ShapeDtypeStruct((nb, M2 * 8, 128), f32),
                   jax.ShapeDtypeStruct((nb, M1, 32), f32),
                   jax.ShapeDtypeStruct((nb, M2, 32), f32)],
    )(nbg1.reshape(nb, M1 * K, 32), coff1, sa1_w2, sa1_b2, c2_w1, c2_b1,
      sa2_w1, sa2_b1, centers1, centers2)

    nbg2 = _make_sc_topk_gather(nb * M2, M1, M2, M1)(
        d2_2.reshape(nb * M2 * M1), g2.reshape(nb * M1 * 32))

    out = pl.pallas_call(
        _p3_body,
        grid=(nb,),
        in_specs=[_batched((M2 * K, 32)), _batched((M2, 32)),
                  _full((32, 128)), _full((128,)),
                  _full((128, 256)), _full((256,))],
        out_specs=_batched((1, 256)),
        out_shape=jax.ShapeDtypeStruct((nb, 1, 256), f32),
    )(nbg2.reshape(nb, M2 * K, 32), coff2, sa2_w2, sa2_b2, mlp_w, mlp_b)

    return out.reshape(nb, 256)


def kernel(x, c1_w1, c1_b1, c1_w2, c1_b2, sa1_w1, sa1_b1, sa1_w2, sa1_b2,
           c2_w1, c2_b1, sa2_w1, sa2_b1, sa2_w2, sa2_b2, mlp_w, mlp_b):
    centers1 = x[:, ::2][:, :M1]
    stride2 = M1 // M2
    centers2 = centers1[:, ::stride2][:, :M2]
    nb = B // 2
    halves = [
        _half_pipeline(x[i * nb:(i + 1) * nb], centers1[i * nb:(i + 1) * nb],
                       centers2[i * nb:(i + 1) * nb], c1_w1, c1_b1, c1_w2,
                       c1_b2, sa1_w1, sa1_b1, sa1_w2, sa1_b2, c2_w1, c2_b1,
                       sa2_w1, sa2_b1, sa2_w2, sa2_b2, mlp_w, mlp_b, nb)
        for i in range(2)
    ]
    return jnp.concatenate(halves, axis=0)


# speedup vs baseline: 1.9866x; 1.0344x over previous
import functools

import jax
import jax.numpy as jnp
from jax import lax
from jax.experimental import pallas as pl
from jax.experimental.pallas import tpu as pltpu
from jax.experimental.pallas import tpu_sc as plsc

B, N1, M1, M2, K = 16, 2048, 1024, 256, 32
_BIG = 1e30
_HI = jax.lax.Precision.DEFAULT


def _dot(a, b):
    return jax.lax.dot_general(a, b, (((1,), (0,)), ((), ())), precision=_HI)


def _dot_t(a, b):
    return jax.lax.dot_general(a, b, (((1,), (1,)), ((), ())), precision=_HI)


def _pairwise_d2(c, x):
    cc = jnp.sum(c * c, axis=-1)
    xx = jnp.sum(x * x, axis=-1)
    return cc[:, None] + xx[None, :] - 2.0 * _dot_t(c, x)


def _topk_mlp_max(d2, g, coff, w2, b2, m, n):
    cols = jax.lax.broadcasted_iota(jnp.int32, (m, n), 1)
    acc0 = jnp.full((m, w2.shape[1]), -_BIG, jnp.float32)

    def body(_, carry):
        d2c, acc = carry
        rmin = jnp.min(d2c, axis=1, keepdims=True)
        cand = jnp.where(d2c == rmin, cols, n)
        amin = jnp.min(cand, axis=1, keepdims=True)
        onehot = (cols == amin)
        d2n = jnp.where(onehot, _BIG, d2c)
        nbg = _dot(onehot.astype(jnp.float32), g)
        h1 = jnp.maximum(nbg - coff, 0.0)
        h2 = jnp.maximum(_dot(h1, w2) + b2, 0.0)
        return d2n, jnp.maximum(acc, h2)

    _, acc = jax.lax.fori_loop(0, K, body, (d2, acc0))
    return acc


def _make_sc_topk_gather(R, N, M, NPTS):
    NC = N // 16
    NG = NC // 16
    NW = 32
    GPW = R // (16 * NW)
    f32, i32 = jnp.float32, jnp.int32
    mesh = plsc.VectorSubcoreMesh(core_axis_name="c", subcore_axis_name="s",
                                  num_cores=2, num_subcores=16)

    @functools.partial(
        pl.kernel,
        out_type=jax.ShapeDtypeStruct((R * K * 32,), f32),
        mesh=mesh,
        compiler_params=pltpu.CompilerParams(needs_layout_passes=False,
                                             use_tc_tiling_on_sc=False),
        scratch_types=[
            pltpu.VMEM((16 * N,), f32),
            pltpu.VMEM((NC * 16,), f32),
            pltpu.VMEM((NG * 16,), f32),
            pltpu.VMEM((16 * K,), i32),
            pltpu.VMEM((16 * K * 32,), f32),
            pltpu.VMEM((NPTS * 32,), f32),
        ],
    )
    def sc_kernel(d2_hbm, g_hbm, out_hbm, dbuf, chmin, l2, idxf, gbuf, gtab):
        wid = lax.axis_index("s") * 2 + lax.axis_index("c")
        lane = lax.iota(i32, 16)
        rowb = lane * N
        bat = (wid * GPW * 16) // M
        pltpu.sync_copy(g_hbm.at[pl.ds(bat * NPTS * 32, NPTS * 32)], gtab)

        def group_body(gi, _):
            row0 = (wid * GPW + gi) * 16
            pltpu.sync_copy(d2_hbm.at[pl.ds(row0 * N, 16 * N)], dbuf)

            def _tree_min(vals):
                while len(vals) > 1:
                    vals = [jnp.minimum(a, b) for a, b in
                            zip(vals[::2], vals[1::2])]
                return vals[0]

            def chunk_body(c, _):
                base = rowb + c * 16
                chmin[pl.ds(c * 16, 16)] = _tree_min(
                    [plsc.load_gather(dbuf, [base + e]) for e in range(16)])
                return 0

            lax.fori_loop(0, NC, chunk_body, 0, unroll=2)

            for q in range(NG):
                l2[pl.ds(q * 16, 16)] = _tree_min(
                    [chmin[pl.ds(q * 256 + e * 16, 16)] for e in range(16)])

            def ext_body(j, _):
                l2v = [l2[pl.ds(q * 16, 16)] for q in range(NG)]
                mv = _tree_min(l2v)
                g2 = _tree_min([jnp.where(l2v[q] == mv, q, NG)
                                for q in range(NG)])
                cb = g2 * 256 + lane
                cvs = [plsc.load_gather(chmin, [cb + e * 16])
                       for e in range(16)]
                c2 = _tree_min([jnp.where(cvs[e] == mv, e, 16)
                                for e in range(16)])
                ch = g2 * 16 + c2
                eb = rowb + ch * 16
                dvs = [plsc.load_gather(dbuf, [eb + e]) for e in range(16)]
                el = _tree_min([jnp.where(dvs[e] == mv, e, 16)
                                for e in range(16)])
                n = ch * 16 + el
                plsc.store_scatter(idxf, [lane * K + j], n)
                plsc.store_scatter(dbuf, [eb + el],
                                   jnp.full((16,), _BIG, f32))
                mv2 = _tree_min([jnp.where(el == e, _BIG, dvs[e])
                                 for e in range(16)])
                plsc.store_scatter(chmin, [ch * 16 + lane], mv2)
                lv = _tree_min([jnp.where(c2 == e, mv2, cvs[e])
                                for e in range(16)])
                plsc.store_scatter(l2, [g2 * 16 + lane], lv)
                return 0

            lax.fori_loop(0, K, ext_body, 0, unroll=False)

            def gat_body(jb, _):
                iv = idxf[pl.ds(jb * 16, 16)]
                for t in range(16):
                    s = iv[t]
                    base = jb * 512 + t * 32
                    gbuf[pl.ds(base, 16)] = gtab[pl.ds(s * 32, 16)]
                    gbuf[pl.ds(base + 16, 16)] = gtab[pl.ds(s * 32 + 16, 16)]
                return 0

            lax.fori_loop(0, K, gat_body, 0, unroll=False)
            pltpu.sync_copy(gbuf, out_hbm.at[pl.ds(row0 * K * 32, 16 * K * 32)])
            return 0

        lax.fori_loop(0, GPW, group_body, 0, unroll=False)

    return sc_kernel


def _p1_body(x_ref, c1_ref, cw1_w1_ref, cw1_b1_ref, c1w2_ref, c1b2_ref,
             sa1w1_ref, sa1b1_ref, d2_ref, g_ref, coff_ref):
    x = x_ref[0]
    c = c1_ref[0]
    feat = jnp.maximum(_dot(x, cw1_w1_ref[...]) + cw1_b1_ref[...], 0.0)
    feat = jnp.maximum(_dot(feat, c1w2_ref[...]) + c1b2_ref[...], 0.0)
    w1 = sa1w1_ref[...]
    g_ref[0] = _dot(x, w1[:3]) + _dot(feat, w1[3:])
    coff_ref[0] = _dot(c, w1[:3]) - sa1b1_ref[...]
    d2_ref[0] = _pairwise_d2(c, x).reshape(M1 * 16, 128)


def _nb_mlp_max(nb, coff, w2, b2, m):
    h1 = jnp.maximum(nb.reshape(m, K, 32) - coff[:, None, :], 0.0)
    h2 = jnp.maximum(_dot(h1.reshape(m * K, 32), w2) + b2, 0.0)
    return jnp.max(h2.reshape(m, K, w2.shape[1]), axis=1)


def _p2_body(nbg_ref, coff_ref, sa1w2_ref, sa1b2_ref,
             c2w1_ref, c2b1_ref, sa2w1_ref, sa2b1_ref, c1_ref, c2_ref,
             d22_ref, g2_ref, coff2_ref):
    feat1 = _nb_mlp_max(nbg_ref[0], coff_ref[0],
                        sa1w2_ref[...], sa1b2_ref[...], M1)
    feat1 = jnp.maximum(_dot(feat1, c2w1_ref[...]) + c2b1_ref[...], 0.0)
    w1 = sa2w1_ref[...]
    c1 = c1_ref[0]
    c2 = c2_ref[0]
    g2_ref[0] = _dot(c1, w1[:3]) + _dot(feat1, w1[3:])
    coff2_ref[0] = _dot(c2, w1[:3]) - sa2b1_ref[...]
    d22_ref[0] = _pairwise_d2(c2, c1).reshape(M2 * 8, 128)


def _p3_body(nbg2_ref, coff2_ref, sa2w2_ref, sa2b2_ref,
             mlpw_ref, mlpb_ref, out_ref):
    feat2 = _nb_mlp_max(nbg2_ref[0], coff2_ref[0],
                        sa2w2_ref[...], sa2b2_ref[...], M2)
    gmax = jnp.max(feat2, axis=0, keepdims=True)
    out_ref[0] = _dot(gmax, mlpw_ref[...]) + mlpb_ref[...]


def _full(shape):
    nd = len(shape)
    return pl.BlockSpec(shape, lambda b: (0,) * nd)


def _batched(shape):
    nd = len(shape)
    return pl.BlockSpec((1,) + shape, lambda b: (b,) + (0,) * nd)


def _half_pipeline(x, centers1, centers2, c1_w1, c1_b1, c1_w2, c1_b2,
                   sa1_w1, sa1_b1, sa1_w2, sa1_b2, c2_w1, c2_b1,
                   sa2_w1, sa2_b1, sa2_w2, sa2_b2, mlp_w, mlp_b, nb):
    f32 = jnp.float32

    d2_1, g1, coff1 = pl.pallas_call(
        _p1_body,
        grid=(nb,),
        in_specs=[_batched((N1, 3)), _batched((M1, 3)),
                  _full((3, 32)), _full((32,)), _full((32, 32)), _full((32,)),
                  _full((35, 32)), _full((32,))],
        out_specs=[_batched((M1 * 16, 128)), _batched((N1, 32)),
                   _batched((M1, 32))],
        out_shape=[jax.ShapeDtypeStruct((nb, M1 * 16, 128), f32),
                   jax.ShapeDtypeStruct((nb, N1, 32), f32),
                   jax.ShapeDtypeStruct((nb, M1, 32), f32)],
    )(x, centers1, c1_w1, c1_b1, c1_w2, c1_b2, sa1_w1, sa1_b1)

    nbg1 = _make_sc_topk_gather(nb * M1, N1, M1, N1)(
        d2_1.reshape(nb * M1 * N1), g1.reshape(nb * N1 * 32))

    d2_2, g2, coff2 = pl.pallas_call(
        _p2_body,
        grid=(nb,),
        in_specs=[_batched((M1 * K, 32)), _batched((M1, 32)),
                  _full((32, 32)), _full((32,)),
                  _full((32, 32)), _full((32,)),
                  _full((35, 32)), _full((32,)),
                  _batched((M1, 3)), _batched((M2, 3))],
        out_specs=[_batched((M2 * 8, 128)), _batched((M1, 32)),
                   _batched((M2, 32))],
        out_shape=[jax.ShapeDtypeStruct((nb, M2 * 8, 128), f32),
                   jax.ShapeDtypeStruct((nb, M1, 32), f32),
                   jax.ShapeDtypeStruct((nb, M2, 32), f32)],
    )(nbg1.reshape(nb, M1 * K, 32), coff1, sa1_w2, sa1_b2, c2_w1, c2_b1,
      sa2_w1, sa2_b1, centers1, centers2)

    nbg2 = _make_sc_topk_gather(nb * M2, M1, M2, M1)(
        d2_2.reshape(nb * M2 * M1), g2.reshape(nb * M1 * 32))

    out = pl.pallas_call(
        _p3_body,
        grid=(nb,),
        in_specs=[_batched((M2 * K, 32)), _batched((M2, 32)),
                  _full((32, 128)), _full((128,)),
                  _full((128, 256)), _full((256,))],
        out_specs=_batched((1, 256)),
        out_shape=jax.ShapeDtypeStruct((nb, 1, 256), f32),
    )(nbg2.reshape(nb, M2 * K, 32), coff2, sa2_w2, sa2_b2, mlp_w, mlp_b)

    return out.reshape(nb, 256)


def kernel(x, c1_w1, c1_b1, c1_w2, c1_b2, sa1_w1, sa1_b1, sa1_w2, sa1_b2,
           c2_w1, c2_b1, sa2_w1, sa2_b1, sa2_w2, sa2_b2, mlp_w, mlp_b):
    centers1 = x[:, ::2][:, :M1]
    stride2 = M1 // M2
    centers2 = centers1[:, ::stride2][:, :M2]
    nb = B // 4
    halves = [
        _half_pipeline(x[i * nb:(i + 1) * nb], centers1[i * nb:(i + 1) * nb],
                       centers2[i * nb:(i + 1) * nb], c1_w1, c1_b1, c1_w2,
                       c1_b2, sa1_w1, sa1_b1, sa1_w2, sa1_b2, c2_w1, c2_b1,
                       sa2_w1, sa2_b1, sa2_w2, sa2_b2, mlp_w, mlp_b, nb)
        for i in range(4)
    ]
    return jnp.concatenate(halves, axis=0)


# speedup vs baseline: 3.1075x; 1.5642x over previous
import functools

import jax
import jax.numpy as jnp
from jax import lax
from jax.experimental import pallas as pl
from jax.experimental.pallas import tpu as pltpu
from jax.experimental.pallas import tpu_sc as plsc

B, N1, M1, M2, K = 16, 2048, 1024, 256, 32
_BIG = 1e30
_HI = jax.lax.Precision.DEFAULT


def _dot(a, b):
    return jax.lax.dot_general(a, b, (((1,), (0,)), ((), ())), precision=_HI)


def _dot_t(a, b):
    return jax.lax.dot_general(a, b, (((1,), (1,)), ((), ())), precision=_HI)


def _pairwise_d2(c, x):
    cc = jnp.sum(c * c, axis=-1)
    xx = jnp.sum(x * x, axis=-1)
    return cc[:, None] + xx[None, :] - 2.0 * _dot_t(c, x)


def _chunk_min(d2, n):
    slabs = [d2[:, 128 * e:128 * (e + 1)] for e in range(n // 128)]
    while len(slabs) > 1:
        slabs = [jnp.minimum(a, b) for a, b in zip(slabs[::2], slabs[1::2])]
    return slabs[0]


def _topk_mlp_max(d2, g, coff, w2, b2, m, n):
    cols = jax.lax.broadcasted_iota(jnp.int32, (m, n), 1)
    acc0 = jnp.full((m, w2.shape[1]), -_BIG, jnp.float32)

    def body(_, carry):
        d2c, acc = carry
        rmin = jnp.min(d2c, axis=1, keepdims=True)
        cand = jnp.where(d2c == rmin, cols, n)
        amin = jnp.min(cand, axis=1, keepdims=True)
        onehot = (cols == amin)
        d2n = jnp.where(onehot, _BIG, d2c)
        nbg = _dot(onehot.astype(jnp.float32), g)
        h1 = jnp.maximum(nbg - coff, 0.0)
        h2 = jnp.maximum(_dot(h1, w2) + b2, 0.0)
        return d2n, jnp.maximum(acc, h2)

    _, acc = jax.lax.fori_loop(0, K, body, (d2, acc0))
    return acc


def _make_sc_topk_gather(R, N, M, NPTS):
    CS = N // 128
    NG = 8
    NW = 32
    GPW = R // (16 * NW)
    f32, i32 = jnp.float32, jnp.int32
    mesh = plsc.VectorSubcoreMesh(core_axis_name="c", subcore_axis_name="s",
                                  num_cores=2, num_subcores=16)

    @functools.partial(
        pl.kernel,
        out_type=jax.ShapeDtypeStruct((R * K * 32,), f32),
        mesh=mesh,
        compiler_params=pltpu.CompilerParams(needs_layout_passes=False,
                                             use_tc_tiling_on_sc=False),
        scratch_types=[
            pltpu.VMEM((16 * N,), f32),
            pltpu.VMEM((16 * 128,), f32),
            pltpu.VMEM((NG * 16,), f32),
            pltpu.VMEM((16 * K,), i32),
            pltpu.VMEM((16 * K * 32,), f32),
            pltpu.VMEM((NPTS * 32,), f32),
        ],
    )
    def sc_kernel(d2_hbm, cm_hbm, g_hbm, out_hbm, dbuf, chmin, l2, idxf,
                  gbuf, gtab):
        wid = lax.axis_index("s") * 2 + lax.axis_index("c")
        lane = lax.iota(i32, 16)
        rowb = lane * N
        rowc = lane * 128
        bat = (wid * GPW * 16) // M
        pltpu.sync_copy(g_hbm.at[pl.ds(bat * NPTS * 32, NPTS * 32)], gtab)

        def _tree_min(vals):
            while len(vals) > 1:
                vals = [jnp.minimum(a, b) for a, b in
                        zip(vals[::2], vals[1::2])]
            return vals[0]

        def group_body(gi, _):
            row0 = (wid * GPW + gi) * 16
            pltpu.sync_copy(d2_hbm.at[pl.ds(row0 * N, 16 * N)], dbuf)
            pltpu.sync_copy(cm_hbm.at[pl.ds(row0 * 128, 16 * 128)], chmin)

            for q in range(NG):
                l2[pl.ds(q * 16, 16)] = _tree_min(
                    [plsc.load_gather(chmin, [rowc + q * 16 + e])
                     for e in range(16)])

            def ext_body(j, _):
                l2v = [l2[pl.ds(q * 16, 16)] for q in range(NG)]
                mv = _tree_min(l2v)
                g2 = _tree_min([jnp.where(l2v[q] == mv, q, NG)
                                for q in range(NG)])
                cb = rowc + g2 * 16
                cvs = [plsc.load_gather(chmin, [cb + e])
                       for e in range(16)]
                c2 = _tree_min([jnp.where(cvs[e] == mv, e, 16)
                                for e in range(16)])
                ch = g2 * 16 + c2
                eb = rowb + ch
                dvs = [plsc.load_gather(dbuf, [eb + 128 * e])
                       for e in range(CS)]
                el = _tree_min([jnp.where(dvs[e] == mv, e, CS)
                                for e in range(CS)])
                n = ch + 128 * el
                plsc.store_scatter(idxf, [lane * K + j], n)
                plsc.store_scatter(dbuf, [rowb + n],
                                   jnp.full((16,), _BIG, f32))
                mv2 = _tree_min([jnp.where(el == e, _BIG, dvs[e])
                                 for e in range(CS)])
                plsc.store_scatter(chmin, [rowc + ch], mv2)
                lv = _tree_min([jnp.where(c2 == e, mv2, cvs[e])
                                for e in range(16)])
                plsc.store_scatter(l2, [g2 * 16 + lane], lv)
                return 0

            lax.fori_loop(0, K, ext_body, 0, unroll=False)

            def gat_body(jb, _):
                iv = idxf[pl.ds(jb * 16, 16)]
                for t in range(16):
                    s = iv[t]
                    base = jb * 512 + t * 32
                    gbuf[pl.ds(base, 16)] = gtab[pl.ds(s * 32, 16)]
                    gbuf[pl.ds(base + 16, 16)] = gtab[pl.ds(s * 32 + 16, 16)]
                return 0

            lax.fori_loop(0, K, gat_body, 0, unroll=False)
            pltpu.sync_copy(gbuf, out_hbm.at[pl.ds(row0 * K * 32, 16 * K * 32)])
            return 0

        lax.fori_loop(0, GPW, group_body, 0, unroll=False)

    return sc_kernel


def _p1_body(x_ref, c1_ref, cw1_w1_ref, cw1_b1_ref, c1w2_ref, c1b2_ref,
             sa1w1_ref, sa1b1_ref, d2_ref, cm_ref, g_ref, coff_ref):
    x = x_ref[0]
    c = c1_ref[0]
    feat = jnp.maximum(_dot(x, cw1_w1_ref[...]) + cw1_b1_ref[...], 0.0)
    feat = jnp.maximum(_dot(feat, c1w2_ref[...]) + c1b2_ref[...], 0.0)
    w1 = sa1w1_ref[...]
    g_ref[0] = _dot(x, w1[:3]) + _dot(feat, w1[3:])
    coff_ref[0] = _dot(c, w1[:3]) - sa1b1_ref[...]
    d2 = _pairwise_d2(c, x)
    d2_ref[0] = d2.reshape(M1 * 16, 128)
    cm_ref[0] = _chunk_min(d2, N1)


def _nb_mlp_max(nb, coff, w2, b2, m):
    h1 = jnp.maximum(nb.reshape(m, K, 32) - coff[:, None, :], 0.0)
    h2 = jnp.maximum(_dot(h1.reshape(m * K, 32), w2) + b2, 0.0)
    return jnp.max(h2.reshape(m, K, w2.shape[1]), axis=1)


def _p2_body(nbg_ref, coff_ref, sa1w2_ref, sa1b2_ref,
             c2w1_ref, c2b1_ref, sa2w1_ref, sa2b1_ref, c1_ref, c2_ref,
             d22_ref, cm2_ref, g2_ref, coff2_ref):
    feat1 = _nb_mlp_max(nbg_ref[0], coff_ref[0],
                        sa1w2_ref[...], sa1b2_ref[...], M1)
    feat1 = jnp.maximum(_dot(feat1, c2w1_ref[...]) + c2b1_ref[...], 0.0)
    w1 = sa2w1_ref[...]
    c1 = c1_ref[0]
    c2 = c2_ref[0]
    g2_ref[0] = _dot(c1, w1[:3]) + _dot(feat1, w1[3:])
    coff2_ref[0] = _dot(c2, w1[:3]) - sa2b1_ref[...]
    d22 = _pairwise_d2(c2, c1)
    d22_ref[0] = d22.reshape(M2 * 8, 128)
    cm2_ref[0] = _chunk_min(d22, M1)


def _p3_body(nbg2_ref, coff2_ref, sa2w2_ref, sa2b2_ref,
             mlpw_ref, mlpb_ref, out_ref):
    feat2 = _nb_mlp_max(nbg2_ref[0], coff2_ref[0],
                        sa2w2_ref[...], sa2b2_ref[...], M2)
    gmax = jnp.max(feat2, axis=0, keepdims=True)
    out_ref[0] = _dot(gmax, mlpw_ref[...]) + mlpb_ref[...]


def _full(shape):
    nd = len(shape)
    return pl.BlockSpec(shape, lambda b: (0,) * nd)


def _batched(shape):
    nd = len(shape)
    return pl.BlockSpec((1,) + shape, lambda b: (b,) + (0,) * nd)


def _half_pipeline(x, centers1, centers2, c1_w1, c1_b1, c1_w2, c1_b2,
                   sa1_w1, sa1_b1, sa1_w2, sa1_b2, c2_w1, c2_b1,
                   sa2_w1, sa2_b1, sa2_w2, sa2_b2, mlp_w, mlp_b, nb):
    f32 = jnp.float32

    d2_1, cm1, g1, coff1 = pl.pallas_call(
        _p1_body,
        grid=(nb,),
        in_specs=[_batched((N1, 3)), _batched((M1, 3)),
                  _full((3, 32)), _full((32,)), _full((32, 32)), _full((32,)),
                  _full((35, 32)), _full((32,))],
        out_specs=[_batched((M1 * 16, 128)), _batched((M1, 128)),
                   _batched((N1, 32)), _batched((M1, 32))],
        out_shape=[jax.ShapeDtypeStruct((nb, M1 * 16, 128), f32),
                   jax.ShapeDtypeStruct((nb, M1, 128), f32),
                   jax.ShapeDtypeStruct((nb, N1, 32), f32),
                   jax.ShapeDtypeStruct((nb, M1, 32), f32)],
    )(x, centers1, c1_w1, c1_b1, c1_w2, c1_b2, sa1_w1, sa1_b1)

    nbg1 = _make_sc_topk_gather(nb * M1, N1, M1, N1)(
        d2_1.reshape(nb * M1 * N1), cm1.reshape(nb * M1 * 128),
        g1.reshape(nb * N1 * 32))

    d2_2, cm2, g2, coff2 = pl.pallas_call(
        _p2_body,
        grid=(nb,),
        in_specs=[_batched((M1 * K, 32)), _batched((M1, 32)),
                  _full((32, 32)), _full((32,)),
                  _full((32, 32)), _full((32,)),
                  _full((35, 32)), _full((32,)),
                  _batched((M1, 3)), _batched((M2, 3))],
        out_specs=[_batched((M2 * 8, 128)), _batched((M2, 128)),
                   _batched((M1, 32)), _batched((M2, 32))],
        out_shape=[jax.ShapeDtypeStruct((nb, M2 * 8, 128), f32),
                   jax.ShapeDtypeStruct((nb, M2, 128), f32),
                   jax.ShapeDtypeStruct((nb, M1, 32), f32),
                   jax.ShapeDtypeStruct((nb, M2, 32), f32)],
    )(nbg1.reshape(nb, M1 * K, 32), coff1, sa1_w2, sa1_b2, c2_w1, c2_b1,
      sa2_w1, sa2_b1, centers1, centers2)

    nbg2 = _make_sc_topk_gather(nb * M2, M1, M2, M1)(
        d2_2.reshape(nb * M2 * M1), cm2.reshape(nb * M2 * 128),
        g2.reshape(nb * M1 * 32))

    out = pl.pallas_call(
        _p3_body,
        grid=(nb,),
        in_specs=[_batched((M2 * K, 32)), _batched((M2, 32)),
                  _full((32, 128)), _full((128,)),
                  _full((128, 256)), _full((256,))],
        out_specs=_batched((1, 256)),
        out_shape=jax.ShapeDtypeStruct((nb, 1, 256), f32),
    )(nbg2.reshape(nb, M2 * K, 32), coff2, sa2_w2, sa2_b2, mlp_w, mlp_b)

    return out.reshape(nb, 256)


def kernel(x, c1_w1, c1_b1, c1_w2, c1_b2, sa1_w1, sa1_b1, sa1_w2, sa1_b2,
           c2_w1, c2_b1, sa2_w1, sa2_b1, sa2_w2, sa2_b2, mlp_w, mlp_b):
    centers1 = x[:, ::2][:, :M1]
    stride2 = M1 // M2
    centers2 = centers1[:, ::stride2][:, :M2]
    nb = B // 4
    halves = [
        _half_pipeline(x[i * nb:(i + 1) * nb], centers1[i * nb:(i + 1) * nb],
                       centers2[i * nb:(i + 1) * nb], c1_w1, c1_b1, c1_w2,
                       c1_b2, sa1_w1, sa1_b1, sa1_w2, sa1_b2, c2_w1, c2_b1,
                       sa2_w1, sa2_b1, sa2_w2, sa2_b2, mlp_w, mlp_b, nb)
        for i in range(4)
    ]
    return jnp.concatenate(halves, axis=0)
